# baseline (speedup 1.0000x reference)
import jax
import jax.numpy as jnp
from jax.experimental import pallas as pl
from jax.experimental.pallas import tpu as pltpu

_HIDDEN = 4096
_EXPERTS = 64
_TOKENS = 8192
_TM = 512
_NT = _TOKENS // _TM


def _router_kernel(h_ref, wt_ref, logits_ref, idx_ref, ew_ref, stat_ref,
                   load_acc, ent_acc):
    i = pl.program_id(0)

    @pl.when(i == 0)
    def _init():
        load_acc[...] = jnp.zeros_like(load_acc)
        ent_acc[0] = 0.0

    h = h_ref[...]
    wt = wt_ref[...]
    logits = jnp.dot(h, wt, preferred_element_type=jnp.float32)
    logits_ref[...] = logits

    m1 = jnp.max(logits, axis=-1, keepdims=True)
    i1 = jnp.argmax(logits, axis=-1).astype(jnp.int32)
    col = jax.lax.broadcasted_iota(jnp.int32, logits.shape, 1)
    masked = jnp.where(col == i1[:, None], -jnp.inf, logits)
    m2 = jnp.max(masked, axis=-1, keepdims=True)
    i2 = jnp.argmax(masked, axis=-1).astype(jnp.int32)
    idx_ref[...] = jnp.concatenate([i1[:, None], i2[:, None]], axis=-1)

    e2 = jnp.exp(m2 - m1)
    denom2 = 1.0 + e2
    ew_ref[...] = jnp.concatenate([1.0 / denom2, e2 / denom2], axis=-1)

    p = jnp.exp(logits - m1)
    denom = jnp.sum(p, axis=-1, keepdims=True)
    probs = p / denom
    load_acc[...] += jnp.sum(probs, axis=0, keepdims=True)
    ent_acc[0] += -jnp.sum(probs * jnp.log(probs + 1e-8))

    @pl.when(i == _NT - 1)
    def _finish():
        load = load_acc[...] / _TOKENS
        mean = jnp.mean(load)
        var = jnp.sum((load - mean) ** 2) / (_EXPERTS - 1)
        stat_ref[0] = var
        stat_ref[1] = ent_acc[0] / _TOKENS


def kernel(hidden_states, router_weight):
    wt = router_weight.T
    logits, idx, ew, stats = pl.pallas_call(
        _router_kernel,
        grid=(_NT,),
        in_specs=[
            pl.BlockSpec((_TM, _HIDDEN), lambda i: (i, 0)),
            pl.BlockSpec((_HIDDEN, _EXPERTS), lambda i: (0, 0)),
        ],
        out_specs=[
            pl.BlockSpec((_TM, _EXPERTS), lambda i: (i, 0)),
            pl.BlockSpec((_TM, 2), lambda i: (i, 0)),
            pl.BlockSpec((_TM, 2), lambda i: (i, 0)),
            pl.BlockSpec(memory_space=pltpu.SMEM),
        ],
        out_shape=[
            jax.ShapeDtypeStruct((_TOKENS, _EXPERTS), jnp.float32),
            jax.ShapeDtypeStruct((_TOKENS, 2), jnp.int32),
            jax.ShapeDtypeStruct((_TOKENS, 2), jnp.float32),
            jax.ShapeDtypeStruct((2,), jnp.float32),
        ],
        scratch_shapes=[
            pltpu.VMEM((1, _EXPERTS), jnp.float32),
            pltpu.SMEM((1,), jnp.float32),
        ],
    )(hidden_states, wt)
    return (logits, idx, ew, stats[0], stats[1])


# speedup vs baseline: 1.0138x; 1.0138x over previous
import jax
import jax.numpy as jnp
from jax.experimental import pallas as pl
from jax.experimental.pallas import tpu as pltpu

_HIDDEN = 4096
_EXPERTS = 64
_TOKENS = 8192
_TM = 1024
_NT = _TOKENS // _TM


def _router_kernel(h_ref, wt_ref, logits_ref, idx_ref, ew_ref, stat_ref,
                   load_acc, ent_acc):
    i = pl.program_id(0)

    @pl.when(i == 0)
    def _init():
        load_acc[...] = jnp.zeros_like(load_acc)
        ent_acc[0] = 0.0

    h = h_ref[...]
    wt = wt_ref[...]
    logits = jnp.dot(h, wt, preferred_element_type=jnp.float32)
    logits_ref[...] = logits

    m1 = jnp.max(logits, axis=-1, keepdims=True)
    i1 = jnp.argmax(logits, axis=-1).astype(jnp.int32)
    col = jax.lax.broadcasted_iota(jnp.int32, logits.shape, 1)
    masked = jnp.where(col == i1[:, None], -jnp.inf, logits)
    m2 = jnp.max(masked, axis=-1, keepdims=True)
    i2 = jnp.argmax(masked, axis=-1).astype(jnp.int32)
    idx_ref[...] = jnp.concatenate([i1[:, None], i2[:, None]], axis=-1)

    e2 = jnp.exp(m2 - m1)
    denom2 = 1.0 + e2
    ew_ref[...] = jnp.concatenate([1.0 / denom2, e2 / denom2], axis=-1)

    p = jnp.exp(logits - m1)
    denom = jnp.sum(p, axis=-1, keepdims=True)
    probs = p / denom
    load_acc[...] += jnp.sum(probs, axis=0, keepdims=True)
    ent_acc[0] += -jnp.sum(probs * jnp.log(probs + 1e-8))

    @pl.when(i == _NT - 1)
    def _finish():
        load = load_acc[...] / _TOKENS
        mean = jnp.mean(load)
        var = jnp.sum((load - mean) ** 2) / (_EXPERTS - 1)
        stat_ref[0] = var
        stat_ref[1] = ent_acc[0] / _TOKENS


def kernel(hidden_states, router_weight):
    wt = router_weight.T
    logits, idx, ew, stats = pl.pallas_call(
        _router_kernel,
        grid=(_NT,),
        in_specs=[
            pl.BlockSpec((_TM, _HIDDEN), lambda i: (i, 0)),
            pl.BlockSpec((_HIDDEN, _EXPERTS), lambda i: (0, 0)),
        ],
        out_specs=[
            pl.BlockSpec((_TM, _EXPERTS), lambda i: (i, 0)),
            pl.BlockSpec((_TM, 2), lambda i: (i, 0)),
            pl.BlockSpec((_TM, 2), lambda i: (i, 0)),
            pl.BlockSpec(memory_space=pltpu.SMEM),
        ],
        out_shape=[
            jax.ShapeDtypeStruct((_TOKENS, _EXPERTS), jnp.float32),
            jax.ShapeDtypeStruct((_TOKENS, 2), jnp.int32),
            jax.ShapeDtypeStruct((_TOKENS, 2), jnp.float32),
            jax.ShapeDtypeStruct((2,), jnp.float32),
        ],
        scratch_shapes=[
            pltpu.VMEM((1, _EXPERTS), jnp.float32),
            pltpu.SMEM((1,), jnp.float32),
        ],
    )(hidden_states, wt)
    return (logits, idx, ew, stats[0], stats[1])


# speedup vs baseline: 1.0721x; 1.0575x over previous
import jax
import jax.numpy as jnp
from jax.experimental import pallas as pl
from jax.experimental.pallas import tpu as pltpu

_HIDDEN = 4096
_EXPERTS = 64
_TOKENS = 8192
_TM = 1024
_NT = _TOKENS // _TM


def _router_kernel(h_ref, w_ref, logits_ref, idx_ref, ew_ref, stat_ref,
                   load_acc, ent_acc):
    i = pl.program_id(0)

    @pl.when(i == 0)
    def _init():
        load_acc[...] = jnp.zeros_like(load_acc)
        ent_acc[0] = 0.0

    h = h_ref[...]
    w = w_ref[...]
    logits = jax.lax.dot_general(
        h, w, (((1,), (1,)), ((), ())),
        preferred_element_type=jnp.float32)
    logits_ref[...] = logits

    m1 = jnp.max(logits, axis=-1, keepdims=True)
    i1 = jnp.argmax(logits, axis=-1).astype(jnp.int32)
    col = jax.lax.broadcasted_iota(jnp.int32, logits.shape, 1)
    masked = jnp.where(col == i1[:, None], -jnp.inf, logits)
    m2 = jnp.max(masked, axis=-1, keepdims=True)
    i2 = jnp.argmax(masked, axis=-1).astype(jnp.int32)
    idx_ref[...] = jnp.concatenate([i1[:, None], i2[:, None]], axis=-1)

    e2 = jnp.exp(m2 - m1)
    denom2 = 1.0 + e2
    ew_ref[...] = jnp.concatenate([1.0 / denom2, e2 / denom2], axis=-1)

    p = jnp.exp(logits - m1)
    denom = jnp.sum(p, axis=-1, keepdims=True)
    probs = p / denom
    load_acc[...] += jnp.sum(probs, axis=0, keepdims=True)
    ent_acc[0] += -jnp.sum(probs * jnp.log(probs + 1e-8))

    @pl.when(i == _NT - 1)
    def _finish():
        load = load_acc[...] / _TOKENS
        mean = jnp.mean(load)
        var = jnp.sum((load - mean) ** 2) / (_EXPERTS - 1)
        stat_ref[0] = var
        stat_ref[1] = ent_acc[0] / _TOKENS


def kernel(hidden_states, router_weight):
    logits, idx, ew, stats = pl.pallas_call(
        _router_kernel,
        grid=(_NT,),
        in_specs=[
            pl.BlockSpec((_TM, _HIDDEN), lambda i: (i, 0)),
            pl.BlockSpec((_EXPERTS, _HIDDEN), lambda i: (0, 0)),
        ],
        out_specs=[
            pl.BlockSpec((_TM, _EXPERTS), lambda i: (i, 0)),
            pl.BlockSpec((_TM, 2), lambda i: (i, 0)),
            pl.BlockSpec((_TM, 2), lambda i: (i, 0)),
            pl.BlockSpec(memory_space=pltpu.SMEM),
        ],
        out_shape=[
            jax.ShapeDtypeStruct((_TOKENS, _EXPERTS), jnp.float32),
            jax.ShapeDtypeStruct((_TOKENS, 2), jnp.int32),
            jax.ShapeDtypeStruct((_TOKENS, 2), jnp.float32),
            jax.ShapeDtypeStruct((2,), jnp.float32),
        ],
        scratch_shapes=[
            pltpu.VMEM((1, _EXPERTS), jnp.float32),
            pltpu.SMEM((1,), jnp.float32),
        ],
    )(hidden_states, router_weight)
    return (logits, idx, ew, stats[0], stats[1])
